# Initial kernel scaffold; baseline (speedup 1.0000x reference)
import jax
import jax.numpy as jnp
from jax import lax
from jax.experimental import pallas as pl
from jax.experimental.pallas import tpu as pltpu
from jax.experimental.pallas import tpu_sc as plsc

N = 10000
E = 160000
H = 256
HALF = 128
G = 64
CUTOFF = 10.0


BN = 1000


def _xf_body(x_ref, w_ref, out_ref):
    r = jnp.dot(x_ref[...], w_ref[...], preferred_element_type=jnp.float32)
    out_ref[0] = r[:, :HALF]
    out_ref[1] = r[:, HALF:]


def _xf(x, lin1_t):
    return pl.pallas_call(
        _xf_body,
        grid=(N // BN,),
        in_specs=[
            pl.BlockSpec((BN, H), lambda i: (i, 0)),
            pl.BlockSpec((H, H), lambda i: (0, 0)),
        ],
        out_specs=pl.BlockSpec((2, BN, HALF), lambda i: (0, i, 0)),
        out_shape=jax.ShapeDtypeStruct((2, N, HALF), jnp.float32),
    )(x, lin1_t)


BE = 1280


def _ssp(v):
    return jax.nn.softplus(v) - jnp.log(2.0)


def _filt_body(ea_ref, el_ref, w1_ref, b1_ref, w2_ref, b2_ref, out_ref):
    h = jnp.dot(ea_ref[...], w1_ref[...], preferred_element_type=jnp.float32)
    h = _ssp(h + b1_ref[...])
    w = jnp.dot(h, w2_ref[...], preferred_element_type=jnp.float32) + b2_ref[...]
    el = el_ref[...]
    c = 0.5 * (jnp.cos(el * (jnp.pi / CUTOFF)) + 1.0)
    c = c * (el <= CUTOFF).astype(jnp.float32) * (el >= 0.0).astype(jnp.float32)
    w = w * c.reshape(BE, 1)
    out_ref[0] = w[:, :HALF]
    out_ref[1] = w[:, HALF:]


def _filt(edge_attr, el2d, w1_t, b1, w2_t, b2):
    return pl.pallas_call(
        _filt_body,
        grid=(E // BE,),
        in_specs=[
            pl.BlockSpec((BE, G), lambda i: (i, 0)),
            pl.BlockSpec((BE // 128, 128), lambda i: (i, 0)),
            pl.BlockSpec((G, H), lambda i: (0, 0)),
            pl.BlockSpec((1, H), lambda i: (0, 0)),
            pl.BlockSpec((H, H), lambda i: (0, 0)),
            pl.BlockSpec((1, H), lambda i: (0, 0)),
        ],
        out_specs=pl.BlockSpec((2, BE, HALF), lambda i: (0, i, 0)),
        out_shape=jax.ShapeDtypeStruct((2, E, HALF), jnp.float32),
    )(edge_attr, el2d, w1_t, b1, w2_t, b2)


B = 80
EPS = E // 16
NCHUNK = EPS // B
NPS = N // 16


def _sc_body(xf2, w2, src2, dst, agg2, sidx, didx, xrows, wrows, accsh, sem):
    c = lax.axis_index("c")
    s = lax.axis_index("s")

    zero16 = jnp.zeros((16,), jnp.float32)

    def zrow(i, carry):
        for j in range(HALF // 16):
            wrows[i, pl.ds(j * 16, 16)] = zero16
        return carry

    lax.fori_loop(0, B, zrow, 0)
    full, rem = divmod(NPS, B)
    for t in range(full):
        pltpu.sync_copy(wrows, accsh.at[pl.ds(s * NPS + t * B, B)])
    if rem:
        pltpu.sync_copy(wrows.at[pl.ds(0, rem)],
                        accsh.at[pl.ds(s * NPS + full * B, rem)])
    plsc.subcore_barrier()

    ebase = s * EPS
    gbase = c * E + ebase

    def chunk(k, carry):
        off = ebase + k * B
        goff = gbase + k * B
        pltpu.sync_copy(src2.at[pl.ds(goff, B)], sidx)
        pltpu.sync_copy(dst.at[pl.ds(off, B)], didx)
        pltpu.async_copy(xf2.at[sidx], xrows, sem).wait()
        pltpu.sync_copy(w2.at[pl.ds(goff, B)], wrows)

        def mul(i, carry2):
            for j in range(HALF // 16):
                sl = pl.ds(j * 16, 16)
                xrows[i, sl] = xrows[i, sl] * wrows[i, sl]
            return carry2

        lax.fori_loop(0, B, mul, 0)
        pltpu.sync_copy(xrows, accsh.at[didx], add=True)
        return carry

    lax.fori_loop(0, NCHUNK, chunk, 0)
    plsc.subcore_barrier()

    pltpu.sync_copy(accsh.at[pl.ds(s * NPS, NPS)],
                    agg2.at[pl.ds(c * N + s * NPS, NPS)])


def _sc_aggregate(xf2, w2, src2, dst):
    mesh = plsc.VectorSubcoreMesh(core_axis_name="c", subcore_axis_name="s")
    return pl.kernel(
        _sc_body,
        out_type=jax.ShapeDtypeStruct((2 * N, HALF), jnp.float32),
        mesh=mesh,
        scratch_types=[
            pltpu.VMEM((B,), jnp.int32),
            pltpu.VMEM((B,), jnp.int32),
            pltpu.VMEM((B, HALF), jnp.float32),
            pltpu.VMEM((B, HALF), jnp.float32),
            pltpu.VMEM_SHARED((N, HALF), jnp.float32),
            pltpu.SemaphoreType.DMA,
        ],
    )(xf2, w2, src2, dst)


def _out_body(agg_ref, w2_ref, b2_ref, w_ref, b_ref, out_ref):
    r = (jnp.dot(agg_ref[0], w2_ref[:HALF], preferred_element_type=jnp.float32)
         + jnp.dot(agg_ref[1], w2_ref[HALF:], preferred_element_type=jnp.float32)
         + b2_ref[...])
    r = _ssp(r)
    out_ref[...] = jnp.dot(r, w_ref[...], preferred_element_type=jnp.float32) + b_ref[...]


def _out_mlp(agg2, lin2_t, lin2_b, lin_t, lin_b):
    return pl.pallas_call(
        _out_body,
        grid=(N // BN,),
        in_specs=[
            pl.BlockSpec((2, BN, HALF), lambda i: (0, i, 0)),
            pl.BlockSpec((H, H), lambda i: (0, 0)),
            pl.BlockSpec((1, H), lambda i: (0, 0)),
            pl.BlockSpec((H, H), lambda i: (0, 0)),
            pl.BlockSpec((1, H), lambda i: (0, 0)),
        ],
        out_specs=pl.BlockSpec((BN, H), lambda i: (i, 0)),
        out_shape=jax.ShapeDtypeStruct((N, H), jnp.float32),
    )(agg2, lin2_t, lin2_b, lin_t, lin_b)


def kernel(x, edge_index, edge_length, edge_attr, lin1_w, mlp_w1, mlp_b1,
           mlp_w2, mlp_b2, lin2_w, lin2_b, lin_w, lin_b):
    src = edge_index[0].astype(jnp.int32)
    dst = edge_index[1].astype(jnp.int32)
    src2 = jnp.concatenate([src, src + N])

    xf2 = _xf(x, lin1_w.T).reshape(2 * N, HALF)
    w2 = _filt(edge_attr, edge_length.reshape(E // 128, 128),
               mlp_w1.T, mlp_b1.reshape(1, H), mlp_w2.T,
               mlp_b2.reshape(1, H)).reshape(2 * E, HALF)
    agg2 = _sc_aggregate(xf2, w2, src2, dst).reshape(2, N, HALF)
    out = _out_mlp(agg2, lin2_t := lin2_w.T, lin2_b.reshape(1, H), lin_w.T,
                   lin_b.reshape(1, H))
    return out


# speedup vs baseline: 1.2300x; 1.2300x over previous
import jax
import jax.numpy as jnp
from jax import lax
from jax.experimental import pallas as pl
from jax.experimental.pallas import tpu as pltpu
from jax.experimental.pallas import tpu_sc as plsc

N = 10000
E = 160000
H = 256
HALF = 128
G = 64
CUTOFF = 10.0


BN = 1000


def _xf_body(x_ref, w_ref, out_ref):
    r = jnp.dot(x_ref[...], w_ref[...], preferred_element_type=jnp.float32)
    out_ref[0] = r[:, :HALF]
    out_ref[1] = r[:, HALF:]


def _xf(x, lin1_t):
    return pl.pallas_call(
        _xf_body,
        grid=(N // BN,),
        in_specs=[
            pl.BlockSpec((BN, H), lambda i: (i, 0)),
            pl.BlockSpec((H, H), lambda i: (0, 0)),
        ],
        out_specs=pl.BlockSpec((2, BN, HALF), lambda i: (0, i, 0)),
        out_shape=jax.ShapeDtypeStruct((2, N, HALF), jnp.float32),
    )(x, lin1_t)


BE = 1280


def _ssp(v):
    return jax.nn.softplus(v) - jnp.log(2.0)


def _filt_body(ea_ref, el_ref, w1_ref, b1_ref, w2_ref, b2_ref, out_ref):
    h = jnp.dot(ea_ref[...], w1_ref[...], preferred_element_type=jnp.float32)
    h = _ssp(h + b1_ref[...])
    w = jnp.dot(h, w2_ref[...], preferred_element_type=jnp.float32) + b2_ref[...]
    el = el_ref[...]
    c = 0.5 * (jnp.cos(el * (jnp.pi / CUTOFF)) + 1.0)
    c = c * (el <= CUTOFF).astype(jnp.float32) * (el >= 0.0).astype(jnp.float32)
    w = w * c
    out_ref[0] = w[:, :HALF]
    out_ref[1] = w[:, HALF:]


def _filt(edge_attr, el2d, w1_t, b1, w2_t, b2):
    return pl.pallas_call(
        _filt_body,
        grid=(E // BE,),
        in_specs=[
            pl.BlockSpec((BE, G), lambda i: (i, 0)),
            pl.BlockSpec((BE, 1), lambda i: (i, 0)),
            pl.BlockSpec((G, H), lambda i: (0, 0)),
            pl.BlockSpec((1, H), lambda i: (0, 0)),
            pl.BlockSpec((H, H), lambda i: (0, 0)),
            pl.BlockSpec((1, H), lambda i: (0, 0)),
        ],
        out_specs=pl.BlockSpec((2, BE, HALF), lambda i: (0, i, 0)),
        out_shape=jax.ShapeDtypeStruct((2, E, HALF), jnp.float32),
    )(edge_attr, el2d, w1_t, b1, w2_t, b2)


B = 80
EPS = E // 16
NCHUNK = EPS // B
SPS = 632
NP = 16 * SPS


def _sc_body(xf2, w2, src2, dst, agg2, sidx, didx, xrows, wrows, accsh, sem):
    c = lax.axis_index("c")
    s = lax.axis_index("s")

    zero16 = jnp.zeros((16,), jnp.float32)

    def zrow(i, carry):
        for j in range(HALF // 16):
            wrows[i, pl.ds(j * 16, 16)] = zero16
        return carry

    lax.fori_loop(0, B, zrow, 0)
    full, rem = divmod(SPS, B)
    for t in range(full):
        pltpu.sync_copy(wrows, accsh.at[pl.ds(s * SPS + t * B, B)])
    if rem:
        pltpu.sync_copy(wrows.at[pl.ds(0, rem)],
                        accsh.at[pl.ds(s * SPS + full * B, rem)])
    plsc.subcore_barrier()

    ebase = s * EPS
    gbase = c * E + ebase

    def chunk(k, carry):
        off = ebase + k * B
        goff = gbase + k * B
        pltpu.sync_copy(src2.at[pl.ds(goff, B)], sidx)
        pltpu.sync_copy(dst.at[pl.ds(off, B)], didx)
        pltpu.async_copy(xf2.at[sidx], xrows, sem).wait()
        pltpu.sync_copy(w2.at[pl.ds(goff, B)], wrows)

        def mul(i, carry2):
            for j in range(HALF // 16):
                sl = pl.ds(j * 16, 16)
                xrows[i, sl] = xrows[i, sl] * wrows[i, sl]
            return carry2

        lax.fori_loop(0, B, mul, 0)
        pltpu.sync_copy(xrows, accsh.at[didx], add=True)
        return carry

    lax.fori_loop(0, NCHUNK, chunk, 0)
    plsc.subcore_barrier()

    pltpu.sync_copy(accsh.at[pl.ds(s * SPS, SPS)],
                    agg2.at[pl.ds(c * NP + s * SPS, SPS)])


def _sc_aggregate(xf2, w2, src2, dst):
    mesh = plsc.VectorSubcoreMesh(core_axis_name="c", subcore_axis_name="s",
                                  num_cores=2, num_subcores=16)
    return pl.kernel(
        _sc_body,
        out_type=jax.ShapeDtypeStruct((2 * NP, HALF), jnp.float32),
        mesh=mesh,
        scratch_types=[
            pltpu.VMEM((B,), jnp.int32),
            pltpu.VMEM((B,), jnp.int32),
            pltpu.VMEM((B, HALF), jnp.float32),
            pltpu.VMEM((B, HALF), jnp.float32),
            pltpu.VMEM_SHARED((NP, HALF), jnp.float32),
            pltpu.SemaphoreType.DMA,
        ],
    )(xf2, w2, src2, dst)


def _out_body(agg_ref, w2_ref, b2_ref, w_ref, b_ref, out_ref):
    r = (jnp.dot(agg_ref[0], w2_ref[:HALF], preferred_element_type=jnp.float32)
         + jnp.dot(agg_ref[1], w2_ref[HALF:], preferred_element_type=jnp.float32)
         + b2_ref[...])
    r = _ssp(r)
    out_ref[...] = jnp.dot(r, w_ref[...], preferred_element_type=jnp.float32) + b_ref[...]


def _out_mlp(agg2, lin2_t, lin2_b, lin_t, lin_b):
    return pl.pallas_call(
        _out_body,
        grid=(N // BN,),
        in_specs=[
            pl.BlockSpec((2, BN, HALF), lambda i: (0, i, 0)),
            pl.BlockSpec((H, H), lambda i: (0, 0)),
            pl.BlockSpec((1, H), lambda i: (0, 0)),
            pl.BlockSpec((H, H), lambda i: (0, 0)),
            pl.BlockSpec((1, H), lambda i: (0, 0)),
        ],
        out_specs=pl.BlockSpec((BN, H), lambda i: (i, 0)),
        out_shape=jax.ShapeDtypeStruct((N, H), jnp.float32),
    )(agg2, lin2_t, lin2_b, lin_t, lin_b)


def kernel(x, edge_index, edge_length, edge_attr, lin1_w, mlp_w1, mlp_b1,
           mlp_w2, mlp_b2, lin2_w, lin2_b, lin_w, lin_b):
    src = edge_index[0].astype(jnp.int32)
    dst = edge_index[1].astype(jnp.int32)
    src2 = jnp.concatenate([src, src + N])

    xf2 = _xf(x, lin1_w.T).reshape(2 * N, HALF)
    w2 = _filt(edge_attr, edge_length.reshape(E, 1),
               mlp_w1.T, mlp_b1.reshape(1, H), mlp_w2.T,
               mlp_b2.reshape(1, H)).reshape(2 * E, HALF)
    agg2 = _sc_aggregate(xf2, w2, src2, dst).reshape(2, NP, HALF)
    out = _out_mlp(agg2, lin2_w.T, lin2_b.reshape(1, H), lin_w.T,
                   lin_b.reshape(1, H))
    return out


# speedup vs baseline: 2.7484x; 2.2345x over previous
import jax
import jax.numpy as jnp
from jax import lax
from jax.experimental import pallas as pl
from jax.experimental.pallas import tpu as pltpu
from jax.experimental.pallas import tpu_sc as plsc

N = 10000
E = 160000
H = 256
HALF = 128
G = 64
CUTOFF = 10.0


BN = 1000


def _xf_body(x_ref, w_ref, out_ref):
    r = jnp.dot(x_ref[...], w_ref[...], preferred_element_type=jnp.float32)
    out_ref[0] = r[:, :HALF]
    out_ref[1] = r[:, HALF:]


def _xf(x, lin1_t):
    return pl.pallas_call(
        _xf_body,
        grid=(N // BN,),
        in_specs=[
            pl.BlockSpec((BN, H), lambda i: (i, 0)),
            pl.BlockSpec((H, H), lambda i: (0, 0)),
        ],
        out_specs=pl.BlockSpec((2, BN, HALF), lambda i: (0, i, 0)),
        out_shape=jax.ShapeDtypeStruct((2, N, HALF), jnp.float32),
    )(x, lin1_t)


BE = 1280


def _ssp(v):
    return jax.nn.softplus(v) - jnp.log(2.0)


def _filt_body(eat_ref, el_ref, w1_ref, b1_ref, w2_ref, b2_ref, out_ref):
    ht = jnp.dot(w1_ref[...], eat_ref[...], preferred_element_type=jnp.float32)
    ht = _ssp(ht + b1_ref[...])
    wt = jnp.dot(w2_ref[...], ht, preferred_element_type=jnp.float32) + b2_ref[...]
    el = el_ref[...]
    c = 0.5 * (jnp.cos(el * (jnp.pi / CUTOFF)) + 1.0)
    c = c * (el <= CUTOFF).astype(jnp.float32) * (el >= 0.0).astype(jnp.float32)
    wt = wt * c
    w = wt.T
    out_ref[0] = w[:, :HALF]
    out_ref[1] = w[:, HALF:]


def _filt(edge_attr_t, el2d, w1, b1, w2, b2):
    return pl.pallas_call(
        _filt_body,
        grid=(E // BE,),
        in_specs=[
            pl.BlockSpec((G, BE), lambda i: (0, i)),
            pl.BlockSpec((1, BE), lambda i: (0, i)),
            pl.BlockSpec((H, G), lambda i: (0, 0)),
            pl.BlockSpec((H, 1), lambda i: (0, 0)),
            pl.BlockSpec((H, H), lambda i: (0, 0)),
            pl.BlockSpec((H, 1), lambda i: (0, 0)),
        ],
        out_specs=pl.BlockSpec((2, BE, HALF), lambda i: (0, i, 0)),
        out_shape=jax.ShapeDtypeStruct((2, E, HALF), jnp.float32),
    )(edge_attr_t, el2d, w1, b1, w2, b2)


B = 80
EPS = E // 16
NCHUNK = EPS // B
SPS = 632
NP = 16 * SPS


def _sc_body(xf2, w2, src2, dst, agg2, sidxs, didx0, didx1, xr0, xr1, wr,
             accsh, semg0, semg1, semw, semd0, semd1):
    c = lax.axis_index("c")
    s = lax.axis_index("s")

    pltpu.sync_copy(src2.at[pl.ds(c * E + s * EPS, EPS)], sidxs)

    zero16 = jnp.zeros((16,), jnp.float32)

    def zrow(i, carry):
        for j in range(HALF // 16):
            wr[i, pl.ds(j * 16, 16)] = zero16
        return carry

    lax.fori_loop(0, B, zrow, 0)
    full, rem = divmod(SPS, B)
    for t in range(full):
        pltpu.sync_copy(wr, accsh.at[pl.ds(s * SPS + t * B, B)])
    if rem:
        pltpu.sync_copy(wr.at[pl.ds(0, rem)],
                        accsh.at[pl.ds(s * SPS + full * B, rem)])
    plsc.subcore_barrier()

    ebase = s * EPS
    gbase = c * E + ebase

    pltpu.async_copy(dst.at[pl.ds(ebase, B)], didx0, semd0)
    pltpu.async_copy(w2.at[pl.ds(gbase, B)], wr, semw)
    pltpu.async_copy(xf2.at[sidxs.at[pl.ds(0, B)]], xr0, semg0)

    def chunk(k, carry):
        def run(xr, semg, di, semd, xr_o, semg_o, di_o, semd_o):
            pltpu.make_async_copy(xf2.at[pl.ds(0, B)], xr, semg).wait()

            @pl.when(k + 1 < NCHUNK)
            def _issue():
                pltpu.async_copy(
                    xf2.at[sidxs.at[pl.ds((k + 1) * B, B)]], xr_o, semg_o)
                pltpu.async_copy(
                    dst.at[pl.ds(ebase + (k + 1) * B, B)], di_o, semd_o)

            pltpu.make_async_copy(w2.at[pl.ds(0, B)], wr, semw).wait()

            def mul(i, carry2):
                for j in range(HALF // 16):
                    sl = pl.ds(j * 16, 16)
                    xr[i, sl] = xr[i, sl] * wr[i, sl]
                return carry2

            lax.fori_loop(0, B, mul, 0)

            @pl.when(k + 1 < NCHUNK)
            def _issue_w():
                pltpu.async_copy(
                    w2.at[pl.ds(gbase + (k + 1) * B, B)], wr, semw)

            pltpu.make_async_copy(dst.at[pl.ds(0, B)], di, semd).wait()
            pltpu.sync_copy(xr, accsh.at[di], add=True)

        @pl.when(k % 2 == 0)
        def _even():
            run(xr0, semg0, didx0, semd0, xr1, semg1, didx1, semd1)

        @pl.when(k % 2 == 1)
        def _odd():
            run(xr1, semg1, didx1, semd1, xr0, semg0, didx0, semd0)

        return carry

    lax.fori_loop(0, NCHUNK, chunk, 0)
    plsc.subcore_barrier()

    pltpu.sync_copy(accsh.at[pl.ds(s * SPS, SPS)],
                    agg2.at[pl.ds(c * NP + s * SPS, SPS)])


def _sc_aggregate(xf2, w2, src2, dst):
    mesh = plsc.VectorSubcoreMesh(core_axis_name="c", subcore_axis_name="s",
                                  num_cores=2, num_subcores=16)
    return pl.kernel(
        _sc_body,
        out_type=jax.ShapeDtypeStruct((2 * NP, HALF), jnp.float32),
        mesh=mesh,
        scratch_types=[
            pltpu.VMEM((EPS,), jnp.int32),
            pltpu.VMEM((B,), jnp.int32),
            pltpu.VMEM((B,), jnp.int32),
            pltpu.VMEM((B, HALF), jnp.float32),
            pltpu.VMEM((B, HALF), jnp.float32),
            pltpu.VMEM((B, HALF), jnp.float32),
            pltpu.VMEM_SHARED((NP, HALF), jnp.float32),
            pltpu.SemaphoreType.DMA,
            pltpu.SemaphoreType.DMA,
            pltpu.SemaphoreType.DMA,
            pltpu.SemaphoreType.DMA,
            pltpu.SemaphoreType.DMA,
        ],
    )(xf2, w2, src2, dst)


def _out_body(agg_ref, w2_ref, b2_ref, w_ref, b_ref, out_ref):
    r = (jnp.dot(agg_ref[0], w2_ref[:HALF], preferred_element_type=jnp.float32)
         + jnp.dot(agg_ref[1], w2_ref[HALF:], preferred_element_type=jnp.float32)
         + b2_ref[...])
    r = _ssp(r)
    out_ref[...] = jnp.dot(r, w_ref[...], preferred_element_type=jnp.float32) + b_ref[...]


def _out_mlp(agg2, lin2_t, lin2_b, lin_t, lin_b):
    return pl.pallas_call(
        _out_body,
        grid=(N // BN,),
        in_specs=[
            pl.BlockSpec((2, BN, HALF), lambda i: (0, i, 0)),
            pl.BlockSpec((H, H), lambda i: (0, 0)),
            pl.BlockSpec((1, H), lambda i: (0, 0)),
            pl.BlockSpec((H, H), lambda i: (0, 0)),
            pl.BlockSpec((1, H), lambda i: (0, 0)),
        ],
        out_specs=pl.BlockSpec((BN, H), lambda i: (i, 0)),
        out_shape=jax.ShapeDtypeStruct((N, H), jnp.float32),
    )(agg2, lin2_t, lin2_b, lin_t, lin_b)


def kernel(x, edge_index, edge_length, edge_attr, lin1_w, mlp_w1, mlp_b1,
           mlp_w2, mlp_b2, lin2_w, lin2_b, lin_w, lin_b):
    src = edge_index[0].astype(jnp.int32)
    dst = edge_index[1].astype(jnp.int32)
    src2 = jnp.concatenate([src, src + N])

    xf2 = _xf(x, lin1_w.T).reshape(2 * N, HALF)
    w2 = _filt(edge_attr.T, edge_length.reshape(1, E),
               mlp_w1, mlp_b1.reshape(H, 1), mlp_w2,
               mlp_b2.reshape(H, 1)).reshape(2 * E, HALF)
    agg2 = _sc_aggregate(xf2, w2, src2, dst).reshape(2, NP, HALF)
    out = _out_mlp(agg2, lin2_w.T, lin2_b.reshape(1, H), lin_w.T,
                   lin_b.reshape(1, H))
    return out
